# Initial kernel scaffold; baseline (speedup 1.0000x reference)
import jax
import jax.numpy as jnp
from jax import lax
from jax.experimental import pallas as pl
from jax.experimental.pallas import tpu as pltpu
from jax.experimental.pallas import tpu_sc as plsc

B, H, W = 8, 512, 512
HW = H * W
N = B * HW
SAME = 0.2

NW = 32
RS2 = N // NW
RS = RS2 // 2
CH = 2048
NCH = HW // CH
NV = CH // 16

_PIB = lax.GatherScatterMode.PROMISE_IN_BOUNDS


def _idx_body(flow_ref, idx_ref):
    b = pl.program_id(0)
    fx = flow_ref[0, 0]
    fy = flow_ref[0, 1]
    zero = (fx == 0.0) & (fy == 0.0)
    fx = jnp.where(zero, 1000.0, fx)
    fy = jnp.where(zero, 1000.0, fy)
    gy = lax.broadcasted_iota(jnp.float32, (H, W), 0)
    gx = lax.broadcasted_iota(jnp.float32, (H, W), 1)
    ty = jnp.round(jnp.clip(gy + fy, 0.0, H - 1.0)).astype(jnp.int32)
    tx = jnp.round(jnp.clip(gx + fx, 0.0, W - 1.0)).astype(jnp.int32)
    idx_ref[0] = b * HW + ty * W + tx


_tc_idx = pl.pallas_call(
    _idx_body,
    grid=(B,),
    in_specs=[pl.BlockSpec((1, 2, H, W), lambda b: (b, 0, 0, 0))],
    out_specs=pl.BlockSpec((1, H, W), lambda b: (b, 0, 0)),
    out_shape=jax.ShapeDtypeStruct((B, H, W), jnp.int32),
)


def _sc_body(idx_hbm, d_hbm, o_hbm, out_hbm, minb, outb, idxc, dc, oc,
             sidx, sd, so):
    c = lax.axis_index("c")
    s = lax.axis_index("s")
    w = s * 2 + c
    lo64 = w * RS2
    img = w // 4
    src0 = img * HW

    iota = lax.broadcasted_iota(jnp.int32, (16,), 0)
    INF = jnp.float32(jnp.inf)

    def seg_ends(k):
        nk = jnp.take(k, jnp.minimum(iota + 1, 15), mode=_PIB)
        return (nk != k) | (iota == 15)

    def seg_reduce(k, v, combine):
        for sh in (1, 2, 4, 8):
            src = jnp.maximum(iota - sh, 0)
            pk = jnp.take(k, src, mode=_PIB)
            pv = jnp.take(v, src, mode=_PIB)
            v = jnp.where(pk == k, combine(v, pv), v)
        return v

    def rmw_min(kv, vv):
        k, v = plsc.sort_key_val(kv, vv)
        v = seg_reduce(k, v, jnp.minimum)
        ends = seg_ends(k)
        off = k - lo64
        cur = plsc.load_gather(minb, [off])
        plsc.store_scatter(minb, [off], jnp.minimum(cur, v), mask=ends)

    def rmw_max(kv, vv, lo_r):
        k, v = plsc.sort_key_val(kv, vv)
        v = seg_reduce(k, v, jnp.maximum)
        ends = seg_ends(k)
        off = k - lo_r
        cur = plsc.load_gather(outb, [off])
        plsc.store_scatter(outb, [off], jnp.maximum(cur, v), mask=ends)

    def initmin(i, x):
        minb[pl.ds(i * 16, 16)] = jnp.full((16,), INF, jnp.float32)
        return x

    lax.fori_loop(0, RS2 // 16, initmin, 0)

    def chunkB(ci, ptr):
        base = src0 + ci * CH
        pltpu.sync_copy(idx_hbm.at[pl.ds(base, CH)], idxc)
        pltpu.sync_copy(d_hbm.at[pl.ds(base, CH)], dc)

        def vec(vi, p):
            iv = idxc[pl.ds(vi * 16, 16)]
            dv = dc[pl.ds(vi * 16, 16)]
            m = (iv >= lo64) & (iv < lo64 + RS2)
            plsc.store_compressed(sidx.at[pl.ds(p, 16)], iv, mask=m)
            plsc.store_compressed(sd.at[pl.ds(p, 16)], dv, mask=m)
            return p + jnp.sum(m.astype(jnp.int32))

        ptr = lax.fori_loop(0, NV, vec, ptr)
        ng = ptr // 16

        def grp(gi, x):
            rmw_min(sidx[pl.ds(gi * 16, 16)], sd[pl.ds(gi * 16, 16)])
            return x

        lax.fori_loop(0, ng, grp, 0)
        ti = sidx[pl.ds(ng * 16, 16)]
        td = sd[pl.ds(ng * 16, 16)]
        sidx[pl.ds(0, 16)] = ti
        sd[pl.ds(0, 16)] = td
        return ptr - ng * 16

    rem = lax.fori_loop(0, NCH, chunkB, 0)
    kv = sidx[pl.ds(0, 16)]
    vv = sd[pl.ds(0, 16)]
    valid = iota < rem
    rmw_min(jnp.where(valid, kv, lo64), jnp.where(valid, vv, INF))

    for r in range(2):
        lo_r = lo64 + r * RS

        def initout(i, x):
            outb[pl.ds(i * 16, 16)] = jnp.full((16,), -INF, jnp.float32)
            return x

        lax.fori_loop(0, RS // 16, initout, 0)

        def chunkC(ci, ptr):
            base = src0 + ci * CH
            pltpu.sync_copy(idx_hbm.at[pl.ds(base, CH)], idxc)
            pltpu.sync_copy(d_hbm.at[pl.ds(base, CH)], dc)
            pltpu.sync_copy(o_hbm.at[pl.ds(base, CH)], oc)

            def vec(vi, p):
                iv = idxc[pl.ds(vi * 16, 16)]
                dv = dc[pl.ds(vi * 16, 16)]
                ov = oc[pl.ds(vi * 16, 16)]
                m = (iv >= lo_r) & (iv < lo_r + RS)
                plsc.store_compressed(sidx.at[pl.ds(p, 16)], iv, mask=m)
                plsc.store_compressed(sd.at[pl.ds(p, 16)], dv, mask=m)
                plsc.store_compressed(so.at[pl.ds(p, 16)], ov, mask=m)
                return p + jnp.sum(m.astype(jnp.int32))

            ptr = lax.fori_loop(0, NV, vec, ptr)
            ng = ptr // 16

            def grp(gi, x):
                kv = sidx[pl.ds(gi * 16, 16)]
                dv = sd[pl.ds(gi * 16, 16)]
                ov = so[pl.ds(gi * 16, 16)]
                mv = plsc.load_gather(minb, [kv - lo64])
                val = jnp.where(dv <= mv + SAME, ov, -INF)
                rmw_max(kv, val, lo_r)
                return x

            lax.fori_loop(0, ng, grp, 0)
            ti = sidx[pl.ds(ng * 16, 16)]
            td = sd[pl.ds(ng * 16, 16)]
            to = so[pl.ds(ng * 16, 16)]
            sidx[pl.ds(0, 16)] = ti
            sd[pl.ds(0, 16)] = td
            so[pl.ds(0, 16)] = to
            return ptr - ng * 16

        rem = lax.fori_loop(0, NCH, chunkC, 0)
        kv = sidx[pl.ds(0, 16)]
        dv = sd[pl.ds(0, 16)]
        ov = so[pl.ds(0, 16)]
        valid = iota < rem
        kv = jnp.where(valid, kv, lo_r)
        mv = plsc.load_gather(minb, [kv - lo64])
        val = jnp.where(valid & (dv <= mv + SAME), ov, -INF)
        rmw_max(kv, val, lo_r)

        def fixup(i, x):
            v = outb[pl.ds(i * 16, 16)]
            outb[pl.ds(i * 16, 16)] = jnp.where(jnp.abs(v) == INF, 0.0, v)
            return x

        lax.fori_loop(0, RS // 16, fixup, 0)
        pltpu.sync_copy(outb, out_hbm.at[pl.ds(lo_r, RS)])


_sc_scatter = pl.kernel(
    _sc_body,
    out_type=jax.ShapeDtypeStruct((N,), jnp.float32),
    mesh=plsc.VectorSubcoreMesh(core_axis_name="c", subcore_axis_name="s"),
    scratch_types=[
        pltpu.VMEM((RS2,), jnp.float32),
        pltpu.VMEM((RS,), jnp.float32),
        pltpu.VMEM((CH,), jnp.int32),
        pltpu.VMEM((CH,), jnp.float32),
        pltpu.VMEM((CH,), jnp.float32),
        pltpu.VMEM((CH + 16,), jnp.int32),
        pltpu.VMEM((CH + 16,), jnp.float32),
        pltpu.VMEM((CH + 16,), jnp.float32),
    ],
)


@jax.jit
def kernel(obj, flow, depth):
    idx = _tc_idx(flow).reshape(N)
    out = _sc_scatter(idx, depth.reshape(N), obj.reshape(N))
    return out.reshape(B, 1, H, W)


# speedup vs baseline: 4.2133x; 4.2133x over previous
import numpy as np
import jax
import jax.numpy as jnp
from jax import lax
from jax.experimental import pallas as pl
from jax.experimental.pallas import tpu as pltpu
from jax.experimental.pallas import tpu_sc as plsc

B, H, W = 8, 512, 512
HW = H * W
N = B * HW
SAME = 0.2

NW = 32
RS2 = N // NW
RS = RS2 // 2
CH = 2048
NCH = HW // CH
NV = CH // 16

_DN = lax.GatherDimensionNumbers(
    offset_dims=(), collapsed_slice_dims=(0,), start_index_map=(0,))

def _vtake(x, i):
    return lax.gather(x, i[:, None], _DN, (1,),
                      mode=lax.GatherScatterMode.PROMISE_IN_BOUNDS)


def _class_combine(k, v, combine, iota):
    out = v
    for sh in range(1, 16):
        rot = (iota + sh) & 15
        pk = _vtake(k, rot)
        pv = _vtake(v, rot)
        out = jnp.where(pk == k, combine(out, pv), out)
    return out


def _idx_body(flow_ref, idx_ref):
    b = pl.program_id(0)
    fx = flow_ref[0, 0]
    fy = flow_ref[0, 1]
    zero = (fx == 0.0) & (fy == 0.0)
    fx = jnp.where(zero, 1000.0, fx)
    fy = jnp.where(zero, 1000.0, fy)
    gy = lax.broadcasted_iota(jnp.int32, (H, W), 0).astype(jnp.float32)
    gx = lax.broadcasted_iota(jnp.int32, (H, W), 1).astype(jnp.float32)
    ty = jnp.round(jnp.clip(gy + fy, 0.0, H - 1.0)).astype(jnp.int32)
    tx = jnp.round(jnp.clip(gx + fx, 0.0, W - 1.0)).astype(jnp.int32)
    idx_ref[0] = b * HW + ty * W + tx


_tc_idx = pl.pallas_call(
    _idx_body,
    grid=(B,),
    in_specs=[pl.BlockSpec((1, 2, H, W), lambda b: (b, 0, 0, 0))],
    out_specs=pl.BlockSpec((1, H, W), lambda b: (b, 0, 0)),
    out_shape=jax.ShapeDtypeStruct((B, H, W), jnp.int32),
)


def _sc_body(idx_hbm, d_hbm, o_hbm, out_hbm, minb, outb, idxc, dc, oc):
    c = lax.axis_index("c")
    s = lax.axis_index("s")
    w = s * 2 + c
    lo = w * RS2
    img = w // 4
    src0 = img * HW

    INF = jnp.float32(jnp.inf)
    iota = lax.broadcasted_iota(jnp.int32, (16,), 0)

    def initmin(i, x):
        minb[pl.ds(i * 16, 16)] = jnp.full((16,), INF, jnp.float32)
        return x

    lax.fori_loop(0, RS2 // 16, initmin, 0, unroll=4)

    def chunkB(ci, x):
        base = src0 + ci * CH
        pltpu.sync_copy(idx_hbm.at[pl.ds(base, CH)], idxc)
        pltpu.sync_copy(d_hbm.at[pl.ds(base, CH)], dc)

        def vec(vi, y):
            iv = idxc[pl.ds(vi * 16, 16)]
            dv = dc[pl.ds(vi * 16, 16)]
            off = iv - lo
            m = (off >= 0) & (off < RS2)
            k = jnp.where(m, off, -1)
            v = jnp.where(m, dv, INF)
            v = _class_combine(k, v, jnp.minimum, iota)
            addr = jnp.where(m, off, 0)
            cur = plsc.load_gather(minb, [addr])
            plsc.store_scatter(minb, [addr], jnp.minimum(cur, v), mask=m)
            return y

        return lax.fori_loop(0, NV, vec, x)

    lax.fori_loop(0, NCH, chunkB, 0)

    for r in range(2):
        lo_r = lo + r * RS

        def initout(i, x):
            outb[pl.ds(i * 16, 16)] = jnp.full((16,), -INF, jnp.float32)
            return x

        lax.fori_loop(0, RS // 16, initout, 0, unroll=4)

        def chunkC(ci, x):
            base = src0 + ci * CH
            pltpu.sync_copy(idx_hbm.at[pl.ds(base, CH)], idxc)
            pltpu.sync_copy(d_hbm.at[pl.ds(base, CH)], dc)
            pltpu.sync_copy(o_hbm.at[pl.ds(base, CH)], oc)

            def vec(vi, y):
                iv = idxc[pl.ds(vi * 16, 16)]
                dv = dc[pl.ds(vi * 16, 16)]
                ov = oc[pl.ds(vi * 16, 16)]
                offr = iv - lo_r
                m = (offr >= 0) & (offr < RS)
                offb = jnp.where(m, iv - lo, 0)
                mv = plsc.load_gather(minb, [offb])
                val = jnp.where(m & (dv <= mv + SAME), ov, -INF)
                k = jnp.where(m, offr, -1)
                val = _class_combine(k, val, jnp.maximum, iota)
                addr = jnp.where(m, offr, 0)
                cur = plsc.load_gather(outb, [addr])
                plsc.store_scatter(outb, [addr], jnp.maximum(cur, val),
                                   mask=m)
                return y

            return lax.fori_loop(0, NV, vec, x)

        lax.fori_loop(0, NCH, chunkC, 0)

        def fixup(i, x):
            v = outb[pl.ds(i * 16, 16)]
            outb[pl.ds(i * 16, 16)] = jnp.where(jnp.abs(v) == INF, 0.0, v)
            return x

        lax.fori_loop(0, RS // 16, fixup, 0, unroll=4)
        pltpu.sync_copy(outb, out_hbm.at[pl.ds(lo_r, RS)])


_sc_scatter = pl.kernel(
    _sc_body,
    out_type=jax.ShapeDtypeStruct((N,), jnp.float32),
    mesh=plsc.VectorSubcoreMesh(core_axis_name="c", subcore_axis_name="s"),
    compiler_params=pltpu.CompilerParams(needs_layout_passes=False),
    scratch_types=[
        pltpu.VMEM((RS2,), jnp.float32),
        pltpu.VMEM((RS,), jnp.float32),
        pltpu.VMEM((CH,), jnp.int32),
        pltpu.VMEM((CH,), jnp.float32),
        pltpu.VMEM((CH,), jnp.float32),
    ],
)


@jax.jit
def kernel(obj, flow, depth):
    idx = _tc_idx(flow).reshape(N)
    out = _sc_scatter(idx, depth.reshape(N), obj.reshape(N))
    return out.reshape(B, 1, H, W)


# speedup vs baseline: 4.2175x; 1.0010x over previous
import numpy as np
import jax
import jax.numpy as jnp
from jax import lax
from jax.experimental import pallas as pl
from jax.experimental.pallas import tpu as pltpu
from jax.experimental.pallas import tpu_sc as plsc

B, H, W = 8, 512, 512
HW = H * W
N = B * HW
SAME = 0.2

NW = 32
RS2 = N // NW
RS = RS2 // 2
CH = 2048
NCH = HW // CH
NV = CH // 16

_DN = lax.GatherDimensionNumbers(
    offset_dims=(), collapsed_slice_dims=(0,), start_index_map=(0,))

def _vtake(x, i):
    return lax.gather(x, i[:, None], _DN, (1,),
                      mode=lax.GatherScatterMode.PROMISE_IN_BOUNDS)


def _class_combine(k, v, combine, rot1):
    out = v
    pk, pv = k, v
    for _ in range(15):
        pk = _vtake(pk, rot1)
        pv = _vtake(pv, rot1)
        out = jnp.where(pk == k, combine(out, pv), out)
    return out


def _idx_body(flow_ref, idx_ref):
    b = pl.program_id(0)
    fx = flow_ref[0, 0]
    fy = flow_ref[0, 1]
    zero = (fx == 0.0) & (fy == 0.0)
    fx = jnp.where(zero, 1000.0, fx)
    fy = jnp.where(zero, 1000.0, fy)
    gy = lax.broadcasted_iota(jnp.int32, (H, W), 0).astype(jnp.float32)
    gx = lax.broadcasted_iota(jnp.int32, (H, W), 1).astype(jnp.float32)
    ty = jnp.round(jnp.clip(gy + fy, 0.0, H - 1.0)).astype(jnp.int32)
    tx = jnp.round(jnp.clip(gx + fx, 0.0, W - 1.0)).astype(jnp.int32)
    idx_ref[0] = b * HW + ty * W + tx


_tc_idx = pl.pallas_call(
    _idx_body,
    grid=(B,),
    in_specs=[pl.BlockSpec((1, 2, H, W), lambda b: (b, 0, 0, 0))],
    out_specs=pl.BlockSpec((1, H, W), lambda b: (b, 0, 0)),
    out_shape=jax.ShapeDtypeStruct((B, H, W), jnp.int32),
)


def _sc_body(idx_hbm, d_hbm, o_hbm, out_hbm, minb, outb, idxc, dc, oc):
    c = lax.axis_index("c")
    s = lax.axis_index("s")
    w = s * 2 + c
    lo = w * RS2
    img = w // 4
    src0 = img * HW

    INF = jnp.float32(jnp.inf)
    iota = lax.broadcasted_iota(jnp.int32, (16,), 0)
    rot1 = (iota + 1) & 15

    def initmin(i, x):
        minb[pl.ds(i * 16, 16)] = jnp.full((16,), INF, jnp.float32)
        return x

    lax.fori_loop(0, RS2 // 16, initmin, 0, unroll=4)

    def chunkB(ci, x):
        base = src0 + ci * CH
        pltpu.sync_copy(idx_hbm.at[pl.ds(base, CH)], idxc)
        pltpu.sync_copy(d_hbm.at[pl.ds(base, CH)], dc)

        def vec(vi, y):
            iv = idxc[pl.ds(vi * 16, 16)]
            dv = dc[pl.ds(vi * 16, 16)]
            off = iv - lo
            m = (off >= 0) & (off < RS2)
            k = jnp.where(m, off, -1)
            v = jnp.where(m, dv, INF)
            v = _class_combine(k, v, jnp.minimum, rot1)
            addr = jnp.where(m, off, 0)
            cur = plsc.load_gather(minb, [addr])
            plsc.store_scatter(minb, [addr], jnp.minimum(cur, v), mask=m)
            return y

        return lax.fori_loop(0, NV, vec, x, unroll=4)

    lax.fori_loop(0, NCH, chunkB, 0)

    for r in range(2):
        lo_r = lo + r * RS

        def initout(i, x):
            outb[pl.ds(i * 16, 16)] = jnp.full((16,), -INF, jnp.float32)
            return x

        lax.fori_loop(0, RS // 16, initout, 0, unroll=4)

        def chunkC(ci, x):
            base = src0 + ci * CH
            pltpu.sync_copy(idx_hbm.at[pl.ds(base, CH)], idxc)
            pltpu.sync_copy(d_hbm.at[pl.ds(base, CH)], dc)
            pltpu.sync_copy(o_hbm.at[pl.ds(base, CH)], oc)

            def vec(vi, y):
                iv = idxc[pl.ds(vi * 16, 16)]
                dv = dc[pl.ds(vi * 16, 16)]
                ov = oc[pl.ds(vi * 16, 16)]
                offr = iv - lo_r
                m = (offr >= 0) & (offr < RS)
                offb = jnp.where(m, iv - lo, 0)
                mv = plsc.load_gather(minb, [offb])
                val = jnp.where(m & (dv <= mv + SAME), ov, -INF)
                k = jnp.where(m, offr, -1)
                val = _class_combine(k, val, jnp.maximum, rot1)
                addr = jnp.where(m, offr, 0)
                cur = plsc.load_gather(outb, [addr])
                plsc.store_scatter(outb, [addr], jnp.maximum(cur, val),
                                   mask=m)
                return y

            return lax.fori_loop(0, NV, vec, x, unroll=4)

        lax.fori_loop(0, NCH, chunkC, 0)

        def fixup(i, x):
            v = outb[pl.ds(i * 16, 16)]
            outb[pl.ds(i * 16, 16)] = jnp.where(jnp.abs(v) == INF, 0.0, v)
            return x

        lax.fori_loop(0, RS // 16, fixup, 0, unroll=4)
        pltpu.sync_copy(outb, out_hbm.at[pl.ds(lo_r, RS)])


_sc_scatter = pl.kernel(
    _sc_body,
    out_type=jax.ShapeDtypeStruct((N,), jnp.float32),
    mesh=plsc.VectorSubcoreMesh(core_axis_name="c", subcore_axis_name="s"),
    compiler_params=pltpu.CompilerParams(needs_layout_passes=False),
    scratch_types=[
        pltpu.VMEM((RS2,), jnp.float32),
        pltpu.VMEM((RS,), jnp.float32),
        pltpu.VMEM((CH,), jnp.int32),
        pltpu.VMEM((CH,), jnp.float32),
        pltpu.VMEM((CH,), jnp.float32),
    ],
)


@jax.jit
def kernel(obj, flow, depth):
    idx = _tc_idx(flow).reshape(N)
    out = _sc_scatter(idx, depth.reshape(N), obj.reshape(N))
    return out.reshape(B, 1, H, W)
